# speedup vs baseline: 1.0125x; 1.0125x over previous
import functools
import math

import jax
import jax.numpy as jnp
import numpy as np
from jax.experimental import pallas as pl
from jax.experimental.pallas import tpu as pltpu

N = 10000
E = 320000
F = 128
N_RBF = 32
DEG_REPEATS = (3, 5, 7)
SPH = 15

EB = 2560
INV_SQRT_F = 1.0 / math.sqrt(F)


def _edge_dense_body(rbf_ref, cc_ref, p_ref, pg_ref, phi_ref,
                     wr1_ref, br1_ref, wr2_ref, br2_ref,
                     sw1_ref, bs1_ref, ws2_ref, bs2_ref,
                     wgr1_ref, bgr1_ref, wgr2_ref, bgr2_ref,
                     alpha_ref, alpha_r_ref):
    rbf = rbf_ref[...]
    cc = cc_ref[...]
    h = jax.nn.silu(jnp.dot(rbf, wr1_ref[...], preferred_element_type=jnp.float32) + br1_ref[...])
    w = jnp.dot(h, wr2_ref[...], preferred_element_type=jnp.float32) + br2_ref[...]
    hs = jax.nn.silu(jnp.dot(cc, sw1_ref[...], preferred_element_type=jnp.float32) + bs1_ref[...])
    w = w + jnp.dot(hs, ws2_ref[...], preferred_element_type=jnp.float32) + bs2_ref[...]
    scale = phi_ref[...] * INV_SQRT_F
    alpha_ref[...] = jnp.sum(p_ref[...] * w, axis=1, keepdims=True) * scale
    g = jax.nn.silu(jnp.dot(rbf, wgr1_ref[...], preferred_element_type=jnp.float32) + bgr1_ref[...])
    wg = jnp.dot(g, wgr2_ref[...], preferred_element_type=jnp.float32) + bgr2_ref[...]
    alpha_r_ref[...] = jnp.sum(pg_ref[...] * wg, axis=1, keepdims=True) * scale


def _edge_dense(rbf, cc, p, pg, phi, wr1, br1, wr2, br2, sw1, bs1, ws2, bs2,
                wgr1, bgr1, wgr2, bgr2):
    grid = E // EB
    eb_spec = lambda w: pl.BlockSpec((EB, w), lambda i: (i, 0))
    full = lambda a: pl.BlockSpec(a.shape, lambda i: (0,) * a.ndim)
    in_specs = [eb_spec(N_RBF), eb_spec(16), eb_spec(F), eb_spec(F), eb_spec(1)]
    ws = (wr1, br1, wr2, br2, sw1, bs1, ws2, bs2, wgr1, bgr1, wgr2, bgr2)
    in_specs += [full(a) for a in ws]
    return pl.pallas_call(
        _edge_dense_body,
        grid=(grid,),
        in_specs=in_specs,
        out_specs=[eb_spec(1), eb_spec(1)],
        out_shape=[jax.ShapeDtypeStruct((E, 1), jnp.float32),
                   jax.ShapeDtypeStruct((E, 1), jnp.float32)],
    )(rbf, cc, p, pg, phi, *ws)


def kernel(sph_ij, chi, idx_j, idx_i, x, rbf, phi_r_cut, Wr1, br1, Wr2, br2,
           Ws1, bs1, Ws2, bs2, Wq, Wk, Wv, Wgr1, bgr1, Wgr2, bgr2,
           Wgs1, bgs1, Wgs2, bgs2, Wqg, Wkg, Wmix, bmix):
    seg = np.zeros((16, 3), np.float32)
    off = 0
    for d, r in enumerate(DEG_REPEATS):
        seg[off:off + r, d] = 1.0
        off += r
    sw1 = jnp.asarray(seg) @ Ws1

    q = x @ Wq
    k = x @ Wk
    v = x @ Wv
    qg = x @ Wqg
    kg = x @ Wkg

    chi16 = jnp.pad(chi, ((0, 0), (0, 1)))
    cc = chi16[idx_i] * chi16[idx_j]
    p = q[idx_i] * k[idx_j]
    pg = qg[idx_i] * kg[idx_j]

    alpha, alpha_r = _edge_dense(
        rbf, cc, p, pg, phi_r_cut,
        Wr1, br1.reshape(1, F), Wr2, br2.reshape(1, F),
        sw1, bs1.reshape(1, F), Ws2, bs2.reshape(1, F),
        Wgr1, bgr1.reshape(1, F), Wgr2, bgr2.reshape(1, F))

    x_local = jax.ops.segment_sum(alpha * v[idx_j], idx_i, num_segments=N)
    chi_local = jax.ops.segment_sum(alpha_r * sph_ij, idx_i, num_segments=N)

    x_skip_1 = x + x_local
    chi_skip_1 = chi + chi_local

    chi2 = chi_skip_1 * chi_skip_1
    segs = jnp.asarray(seg[:15, :])
    inv = chi2 @ segs
    a = jnp.concatenate([x_skip_1, inv], axis=-1) @ Wmix + bmix
    delta_x = a[:, :F]
    coeff = a[:, F:]
    delta_chi = (coeff @ segs.T) * chi_skip_1
    return (x_skip_1 + delta_x, chi_skip_1 + delta_chi)
